# baseline (device time: 64705 ns/iter reference)
import jax
import jax.numpy as jnp
from jax import lax
from jax.experimental import pallas as pl
from jax.experimental.pallas import tpu as pltpu

N_DEV = 4
SQ = 1024
SKV = 1024
HQ = 8
DH = 128
D = HQ * DH
HALF = SQ // 2
SCALE = 0.08838834764831843


def kernel(x, Wq, K_ext, V_ext, Wo):
    def body(x_ref, wq_ref, k_ref, v_ref, wo_ref, out_ref,
             acc_ref, l_ref, pack1_ref, pack2_ref, lp1_ref, lp2_ref,
             rctx_ref, rl_ref, send_sems, recv_sems):
        p = lax.axis_index("i")
        p_y = p ^ 1
        p_x = 3 - p

        xb = x_ref[0].astype(jnp.bfloat16)
        wqb = wq_ref[...].astype(jnp.bfloat16)
        q = lax.dot_general(xb, wqb, (((1,), (0,)), ((), ())),
                            preferred_element_type=jnp.float32)
        qb = (q * SCALE).astype(jnp.bfloat16)
        kb = k_ref[0].reshape(SKV, D).astype(jnp.bfloat16)
        vb = v_ref[0].reshape(SKV, D).astype(jnp.bfloat16)

        qi = lax.broadcasted_iota(jnp.int32, (SQ, SKV), 0)
        ki = lax.broadcasted_iota(jnp.int32, (SQ, SKV), 1)
        mask = ((qi // 64) % 4) == ((ki // 64) % 4)

        for h in range(HQ):
            cols = pl.ds(h * DH, DH)
            qh = qb[:, h * DH:(h + 1) * DH]
            kh = kb[:, h * DH:(h + 1) * DH]
            vh = vb[:, h * DH:(h + 1) * DH]
            s = lax.dot_general(qh, kh, (((1,), (1,)), ((), ())),
                                preferred_element_type=jnp.float32)
            w = jnp.where(mask, jnp.exp(s), 0.0)
            l_ref[:, h] = w.sum(axis=1)
            acc_ref[:, cols] = lax.dot_general(
                w.astype(jnp.bfloat16), vh, (((1,), (0,)), ((), ())),
                preferred_element_type=jnp.float32)

        pack1_ref[...] = acc_ref[...].astype(jnp.bfloat16)
        lp1_ref[...] = l_ref[...]

        bar = pltpu.get_barrier_semaphore()
        for nbr in (p_x, p_y):
            pl.semaphore_signal(bar, inc=1, device_id=(nbr,),
                                device_id_type=pl.DeviceIdType.MESH)
        pl.semaphore_wait(bar, 2)

        def exchange(src, dst, partner, sem_idx):
            return pltpu.make_async_remote_copy(
                src_ref=src, dst_ref=dst,
                send_sem=send_sems.at[sem_idx],
                recv_sem=recv_sems.at[sem_idx],
                device_id=(partner,),
                device_id_type=pl.DeviceIdType.MESH,
            )

        top = pl.ds(0, HALF)
        bot = pl.ds(HALF, HALF)

        r1 = [
            exchange(pack1_ref.at[top], rctx_ref.at[0], p_x, 0),
            exchange(pack1_ref.at[bot], rctx_ref.at[1], p_y, 1),
            exchange(lp1_ref.at[top], rl_ref.at[0], p_x, 2),
            exchange(lp1_ref.at[bot], rl_ref.at[1], p_y, 3),
        ]
        for rdma in r1:
            rdma.start()
        for rdma in r1:
            rdma.wait()

        acc_ref[top, :] = acc_ref[top, :] + rctx_ref[0].astype(jnp.float32)
        acc_ref[bot, :] = acc_ref[bot, :] + rctx_ref[1].astype(jnp.float32)
        l_ref[top, :] = l_ref[top, :] + rl_ref[0]
        l_ref[bot, :] = l_ref[bot, :] + rl_ref[1]

        pack2_ref[...] = acc_ref[...].astype(jnp.bfloat16)
        lp2_ref[...] = l_ref[...]

        r2 = [
            exchange(pack2_ref.at[top], rctx_ref.at[2], p_y, 4),
            exchange(pack2_ref.at[bot], rctx_ref.at[3], p_x, 5),
            exchange(lp2_ref.at[top], rl_ref.at[2], p_y, 6),
            exchange(lp2_ref.at[bot], rl_ref.at[3], p_x, 7),
        ]
        for rdma in r2:
            rdma.start()
        for rdma in r2:
            rdma.wait()

        acc_ref[top, :] = acc_ref[top, :] + rctx_ref[2].astype(jnp.float32)
        acc_ref[bot, :] = acc_ref[bot, :] + rctx_ref[3].astype(jnp.float32)
        l_ref[top, :] = l_ref[top, :] + rl_ref[2]
        l_ref[bot, :] = l_ref[bot, :] + rl_ref[3]

        ctx = (acc_ref[...].reshape(SQ, HQ, DH)
               / l_ref[...][:, :, None]).reshape(SQ, D)
        wob = wo_ref[...].astype(jnp.bfloat16)
        out_ref[0] = lax.dot_general(
            ctx.astype(jnp.bfloat16), wob, (((1,), (0,)), ((), ())),
            preferred_element_type=jnp.float32)

    return pl.pallas_call(
        body,
        out_shape=jax.ShapeDtypeStruct((1, SQ, D), jnp.float32),
        in_specs=[pl.BlockSpec(memory_space=pltpu.VMEM)] * 5,
        out_specs=pl.BlockSpec(memory_space=pltpu.VMEM),
        scratch_shapes=[
            pltpu.VMEM((SQ, D), jnp.float32),
            pltpu.VMEM((SQ, HQ), jnp.float32),
            pltpu.VMEM((SQ, D), jnp.bfloat16),
            pltpu.VMEM((SQ, D), jnp.bfloat16),
            pltpu.VMEM((SQ, HQ), jnp.float32),
            pltpu.VMEM((SQ, HQ), jnp.float32),
            pltpu.VMEM((4, HALF, D), jnp.bfloat16),
            pltpu.VMEM((4, HALF, HQ), jnp.float32),
            pltpu.SemaphoreType.DMA((8,)),
            pltpu.SemaphoreType.DMA((8,)),
        ],
        compiler_params=pltpu.CompilerParams(collective_id=0),
    )(x, Wq, K_ext, V_ext, Wo)


# device time: 57388 ns/iter; 1.1275x vs baseline; 1.1275x over previous
import jax
import jax.numpy as jnp
from jax import lax
from jax.experimental import pallas as pl
from jax.experimental.pallas import tpu as pltpu

N_DEV = 4
SQ = 1024
SKV = 1024
HQ = 8
DH = 128
D = HQ * DH
HALF = SQ // 2
SCALE = 0.08838834764831843


def kernel(x, Wq, K_ext, V_ext, Wo):
    def body(x_ref, wq_ref, k_ref, v_ref, wo_ref, out_ref,
             acc_ref, l_ref, pack1_ref, pack2_ref, lp1_ref, lp2_ref,
             rctx_ref, rl_ref, send_sems, recv_sems):
        p = lax.axis_index("i")
        p_y = p ^ 1
        p_x = 3 - p

        def group_rows(m):
            return m.reshape(4, 4, 64, D).transpose(1, 0, 2, 3).reshape(SQ, D)

        xg = group_rows(x_ref[0].astype(jnp.bfloat16))
        wqb = wq_ref[...].astype(jnp.bfloat16)
        q = lax.dot_general(xg, wqb, (((1,), (0,)), ((), ())),
                            preferred_element_type=jnp.float32)
        qb = (q * SCALE).astype(jnp.bfloat16)
        kb = group_rows(k_ref[0].reshape(SKV, D).astype(jnp.bfloat16))
        vb = group_rows(v_ref[0].reshape(SKV, D).astype(jnp.bfloat16))

        G = SQ // 4
        for r in range(4):
            rows = pl.ds(r * G, G)
            for h in range(HQ):
                cols = pl.ds(h * DH, DH)
                qh = qb[r * G:(r + 1) * G, h * DH:(h + 1) * DH]
                kh = kb[r * G:(r + 1) * G, h * DH:(h + 1) * DH]
                vh = vb[r * G:(r + 1) * G, h * DH:(h + 1) * DH]
                s = lax.dot_general(qh, kh, (((1,), (1,)), ((), ())),
                                    preferred_element_type=jnp.float32)
                w = jnp.exp(s)
                l_ref[rows, h] = w.sum(axis=1)
                acc_ref[rows, cols] = lax.dot_general(
                    w.astype(jnp.bfloat16), vh, (((1,), (0,)), ((), ())),
                    preferred_element_type=jnp.float32)

        pack1_ref[...] = acc_ref[...].astype(jnp.bfloat16)
        lp1_ref[...] = l_ref[...]

        bar = pltpu.get_barrier_semaphore()
        for nbr in (p_x, p_y):
            pl.semaphore_signal(bar, inc=1, device_id=(nbr,),
                                device_id_type=pl.DeviceIdType.MESH)
        pl.semaphore_wait(bar, 2)

        def exchange(src, dst, partner, sem_idx):
            return pltpu.make_async_remote_copy(
                src_ref=src, dst_ref=dst,
                send_sem=send_sems.at[sem_idx],
                recv_sem=recv_sems.at[sem_idx],
                device_id=(partner,),
                device_id_type=pl.DeviceIdType.MESH,
            )

        top = pl.ds(0, HALF)
        bot = pl.ds(HALF, HALF)

        r1 = [
            exchange(pack1_ref.at[top], rctx_ref.at[0], p_x, 0),
            exchange(pack1_ref.at[bot], rctx_ref.at[1], p_y, 1),
            exchange(lp1_ref.at[top], rl_ref.at[0], p_x, 2),
            exchange(lp1_ref.at[bot], rl_ref.at[1], p_y, 3),
        ]
        for rdma in r1:
            rdma.start()
        for rdma in r1:
            rdma.wait()

        acc_ref[top, :] = acc_ref[top, :] + rctx_ref[0].astype(jnp.float32)
        acc_ref[bot, :] = acc_ref[bot, :] + rctx_ref[1].astype(jnp.float32)
        l_ref[top, :] = l_ref[top, :] + rl_ref[0]
        l_ref[bot, :] = l_ref[bot, :] + rl_ref[1]

        pack2_ref[...] = acc_ref[...].astype(jnp.bfloat16)
        lp2_ref[...] = l_ref[...]

        r2 = [
            exchange(pack2_ref.at[top], rctx_ref.at[2], p_y, 4),
            exchange(pack2_ref.at[bot], rctx_ref.at[3], p_x, 5),
            exchange(lp2_ref.at[top], rl_ref.at[2], p_y, 6),
            exchange(lp2_ref.at[bot], rl_ref.at[3], p_x, 7),
        ]
        for rdma in r2:
            rdma.start()
        for rdma in r2:
            rdma.wait()

        acc_ref[top, :] = acc_ref[top, :] + rctx_ref[2].astype(jnp.float32)
        acc_ref[bot, :] = acc_ref[bot, :] + rctx_ref[3].astype(jnp.float32)
        l_ref[top, :] = l_ref[top, :] + rl_ref[2]
        l_ref[bot, :] = l_ref[bot, :] + rl_ref[3]

        ctx = (acc_ref[...].reshape(SQ, HQ, DH)
               / l_ref[...][:, :, None]).reshape(SQ, D)
        wob = wo_ref[...].astype(jnp.bfloat16)
        out_g = lax.dot_general(
            ctx.astype(jnp.bfloat16), wob, (((1,), (0,)), ((), ())),
            preferred_element_type=jnp.float32)
        out_ref[0] = out_g.reshape(4, 4, 64, D).transpose(1, 0, 2, 3).reshape(SQ, D)

    return pl.pallas_call(
        body,
        out_shape=jax.ShapeDtypeStruct((1, SQ, D), jnp.float32),
        in_specs=[pl.BlockSpec(memory_space=pltpu.VMEM)] * 5,
        out_specs=pl.BlockSpec(memory_space=pltpu.VMEM),
        scratch_shapes=[
            pltpu.VMEM((SQ, D), jnp.float32),
            pltpu.VMEM((SQ, HQ), jnp.float32),
            pltpu.VMEM((SQ, D), jnp.bfloat16),
            pltpu.VMEM((SQ, D), jnp.bfloat16),
            pltpu.VMEM((SQ, HQ), jnp.float32),
            pltpu.VMEM((SQ, HQ), jnp.float32),
            pltpu.VMEM((4, HALF, D), jnp.bfloat16),
            pltpu.VMEM((4, HALF, HQ), jnp.float32),
            pltpu.SemaphoreType.DMA((8,)),
            pltpu.SemaphoreType.DMA((8,)),
        ],
        compiler_params=pltpu.CompilerParams(collective_id=0),
    )(x, Wq, K_ext, V_ext, Wo)


# device time: 56982 ns/iter; 1.1355x vs baseline; 1.0071x over previous
import jax
import jax.numpy as jnp
from jax import lax
from jax.experimental import pallas as pl
from jax.experimental.pallas import tpu as pltpu

N_DEV = 4
SQ = 1024
SKV = 1024
HQ = 8
DH = 128
D = HQ * DH
HALF = SQ // 2
G = SQ // 4
SCALE = 0.08838834764831843


def kernel(x, Wq, K_ext, V_ext, Wo):
    def body(x_ref, wq_ref, k_ref, v_ref, wo_ref, out_ref,
             acc_ref, l_ref, pack1_ref, pack2_ref, lp1_ref, lp2_ref,
             rctx_ref, rl_ref, send_sems, recv_sems):
        p = lax.axis_index("i")
        p_y = p ^ 1
        p_x = 3 - p

        bar = pltpu.get_barrier_semaphore()
        for nbr in (p_x, p_y):
            pl.semaphore_signal(bar, inc=1, device_id=(nbr,),
                                device_id_type=pl.DeviceIdType.MESH)
        pl.semaphore_wait(bar, 2)

        def exchange(src, dst, partner, sem_idx):
            return pltpu.make_async_remote_copy(
                src_ref=src, dst_ref=dst,
                send_sem=send_sems.at[sem_idx],
                recv_sem=recv_sems.at[sem_idx],
                device_id=(partner,),
                device_id_type=pl.DeviceIdType.MESH,
            )

        top = pl.ds(0, HALF)
        bot = pl.ds(HALF, HALF)

        def group_rows(m):
            return m.reshape(4, 4, 64, D).transpose(1, 0, 2, 3).reshape(SQ, D)

        xg = group_rows(x_ref[0].astype(jnp.bfloat16))
        wqb = wq_ref[...].astype(jnp.bfloat16)
        q = lax.dot_general(xg, wqb, (((1,), (0,)), ((), ())),
                            preferred_element_type=jnp.float32)
        qb = (q * SCALE).astype(jnp.bfloat16)
        kb = group_rows(k_ref[0].reshape(SKV, D).astype(jnp.bfloat16))
        vb = group_rows(v_ref[0].reshape(SKV, D).astype(jnp.bfloat16))

        def attn_group(r):
            rows = pl.ds(r * G, G)
            for h in range(HQ):
                cols = pl.ds(h * DH, DH)
                qh = qb[r * G:(r + 1) * G, h * DH:(h + 1) * DH]
                kh = kb[r * G:(r + 1) * G, h * DH:(h + 1) * DH]
                vh = vb[r * G:(r + 1) * G, h * DH:(h + 1) * DH]
                s = lax.dot_general(qh, kh, (((1,), (1,)), ((), ())),
                                    preferred_element_type=jnp.float32)
                w = jnp.exp(s)
                l_ref[rows, h] = w.sum(axis=1)
                ctx = lax.dot_general(
                    w.astype(jnp.bfloat16), vh, (((1,), (0,)), ((), ())),
                    preferred_element_type=jnp.float32)
                acc_ref[rows, cols] = ctx
                pack1_ref[rows, cols] = ctx.astype(jnp.bfloat16)

        attn_group(0)
        attn_group(1)
        lp1_ref[top, :] = l_ref[top, :]
        r1a = [exchange(pack1_ref.at[top], rctx_ref.at[0], p_x, 0),
               exchange(lp1_ref.at[top], rl_ref.at[0], p_x, 2)]
        for rdma in r1a:
            rdma.start()

        attn_group(2)
        attn_group(3)
        lp1_ref[bot, :] = l_ref[bot, :]
        r1b = [exchange(pack1_ref.at[bot], rctx_ref.at[1], p_y, 1),
               exchange(lp1_ref.at[bot], rl_ref.at[1], p_y, 3)]
        for rdma in r1b:
            rdma.start()

        for rdma in r1a:
            rdma.wait()
        acc_ref[top, :] = acc_ref[top, :] + rctx_ref[0].astype(jnp.float32)
        l_ref[top, :] = l_ref[top, :] + rl_ref[0]
        pack2_ref[top, :] = acc_ref[top, :].astype(jnp.bfloat16)
        lp2_ref[top, :] = l_ref[top, :]
        r2a = [exchange(pack2_ref.at[top], rctx_ref.at[2], p_y, 4),
               exchange(lp2_ref.at[top], rl_ref.at[2], p_y, 6)]
        for rdma in r2a:
            rdma.start()

        for rdma in r1b:
            rdma.wait()
        acc_ref[bot, :] = acc_ref[bot, :] + rctx_ref[1].astype(jnp.float32)
        l_ref[bot, :] = l_ref[bot, :] + rl_ref[1]
        pack2_ref[bot, :] = acc_ref[bot, :].astype(jnp.bfloat16)
        lp2_ref[bot, :] = l_ref[bot, :]
        r2b = [exchange(pack2_ref.at[bot], rctx_ref.at[3], p_x, 5),
               exchange(lp2_ref.at[bot], rl_ref.at[3], p_x, 7)]
        for rdma in r2b:
            rdma.start()

        wob = wo_ref[...].astype(jnp.bfloat16)

        for rdma in r2a:
            rdma.wait()
        acc_t = acc_ref[top, :] + rctx_ref[2].astype(jnp.float32)
        l_t = l_ref[top, :] + rl_ref[2]
        ctx_t = (acc_t.reshape(HALF, HQ, DH) / l_t[:, :, None]).reshape(HALF, D)
        out_t = lax.dot_general(
            ctx_t.astype(jnp.bfloat16), wob, (((1,), (0,)), ((), ())),
            preferred_element_type=jnp.float32)
        ot = out_t.reshape(2, 4, 64, D).transpose(1, 0, 2, 3)
        for a in range(4):
            out_ref[0, 256 * a:256 * a + 128, :] = ot[a].reshape(128, D)

        for rdma in r2b:
            rdma.wait()
        acc_b = acc_ref[bot, :] + rctx_ref[3].astype(jnp.float32)
        l_b = l_ref[bot, :] + rl_ref[3]
        ctx_b = (acc_b.reshape(HALF, HQ, DH) / l_b[:, :, None]).reshape(HALF, D)
        out_b = lax.dot_general(
            ctx_b.astype(jnp.bfloat16), wob, (((1,), (0,)), ((), ())),
            preferred_element_type=jnp.float32)
        ob = out_b.reshape(2, 4, 64, D).transpose(1, 0, 2, 3)
        for a in range(4):
            out_ref[0, 256 * a + 128:256 * (a + 1), :] = ob[a].reshape(128, D)

    return pl.pallas_call(
        body,
        out_shape=jax.ShapeDtypeStruct((1, SQ, D), jnp.float32),
        in_specs=[pl.BlockSpec(memory_space=pltpu.VMEM)] * 5,
        out_specs=pl.BlockSpec(memory_space=pltpu.VMEM),
        scratch_shapes=[
            pltpu.VMEM((SQ, D), jnp.float32),
            pltpu.VMEM((SQ, HQ), jnp.float32),
            pltpu.VMEM((SQ, D), jnp.bfloat16),
            pltpu.VMEM((SQ, D), jnp.bfloat16),
            pltpu.VMEM((SQ, HQ), jnp.float32),
            pltpu.VMEM((SQ, HQ), jnp.float32),
            pltpu.VMEM((4, HALF, D), jnp.bfloat16),
            pltpu.VMEM((4, HALF, HQ), jnp.float32),
            pltpu.SemaphoreType.DMA((8,)),
            pltpu.SemaphoreType.DMA((8,)),
        ],
        compiler_params=pltpu.CompilerParams(collective_id=0),
    )(x, Wq, K_ext, V_ext, Wo)


# device time: 52317 ns/iter; 1.2368x vs baseline; 1.0892x over previous
import jax
import jax.numpy as jnp
from jax import lax
from jax.experimental import pallas as pl
from jax.experimental.pallas import tpu as pltpu

N_DEV = 4
SQ = 1024
SKV = 1024
HQ = 8
DH = 128
D = HQ * DH
G = SQ // 4
SCALE = 0.08838834764831843


def kernel(x, Wq, K_ext, V_ext, Wo):
    def body(x_ref, wq_ref, k_hbm, v_hbm, wo_hbm, out_ref,
             kv_ref, vv_ref, wo_ref, acc_ref, l_ref, pack1_ref, pack2_ref,
             lp_ref, rctx_ref, rl_ref, load_sems, send_sems, recv_sems):
        p = lax.axis_index("i")
        p_y = p ^ 1
        p_x = 3 - p

        ldk = pltpu.make_async_copy(k_hbm, kv_ref, load_sems.at[0])
        ldv = pltpu.make_async_copy(v_hbm, vv_ref, load_sems.at[1])
        ldw = pltpu.make_async_copy(wo_hbm, wo_ref, load_sems.at[2])
        ldk.start()
        ldv.start()
        ldw.start()

        bar = pltpu.get_barrier_semaphore()
        for nbr in (p_x, p_y):
            pl.semaphore_signal(bar, inc=1, device_id=(nbr,),
                                device_id_type=pl.DeviceIdType.MESH)
        pl.semaphore_wait(bar, 2)

        def exchange(src, dst, partner, sem_idx):
            return pltpu.make_async_remote_copy(
                src_ref=src, dst_ref=dst,
                send_sem=send_sems.at[sem_idx],
                recv_sem=recv_sems.at[sem_idx],
                device_id=(partner,),
                device_id_type=pl.DeviceIdType.MESH,
            )

        def group_rows(m):
            return m.reshape(4, 4, 64, D).transpose(1, 0, 2, 3).reshape(SQ, D)

        xg = group_rows(x_ref[0].astype(jnp.bfloat16))
        wqb = wq_ref[...].astype(jnp.bfloat16)

        ldk.wait()
        ldv.wait()
        kb = group_rows(kv_ref[0].reshape(SKV, D).astype(jnp.bfloat16))
        vb = group_rows(vv_ref[0].reshape(SKV, D).astype(jnp.bfloat16))

        link1 = [p_x, p_y, p_x, p_y]
        link2 = [p_y, p_x, p_y, p_x]
        r1 = [None] * 4
        r2 = [None] * 4

        def rows(q):
            return pl.ds(q * G, G)

        for q in range(4):
            qq = lax.dot_general(xg[q * G:(q + 1) * G, :], wqb,
                                 (((1,), (0,)), ((), ())),
                                 preferred_element_type=jnp.float32)
            qqb = (qq * SCALE).astype(jnp.bfloat16)
            for h in range(HQ):
                cols = pl.ds(h * DH, DH)
                kh = kb[q * G:(q + 1) * G, h * DH:(h + 1) * DH]
                vh = vb[q * G:(q + 1) * G, h * DH:(h + 1) * DH]
                s = lax.dot_general(qqb[:, h * DH:(h + 1) * DH], kh,
                                    (((1,), (1,)), ((), ())),
                                    preferred_element_type=jnp.float32)
                w = jnp.exp(s)
                l_ref[rows(q), h] = w.sum(axis=1)
                ctx = lax.dot_general(
                    w.astype(jnp.bfloat16), vh, (((1,), (0,)), ((), ())),
                    preferred_element_type=jnp.float32)
                acc_ref[rows(q), cols] = ctx
                pack1_ref[rows(q), cols] = ctx.astype(jnp.bfloat16)
            lp_ref[0, rows(q), :] = l_ref[rows(q), :]
            r1[q] = [exchange(pack1_ref.at[rows(q)], rctx_ref.at[q],
                              link1[q], q),
                     exchange(lp_ref.at[0, rows(q)], rl_ref.at[q],
                              link1[q], 8 + q)]
            for rdma in r1[q]:
                rdma.start()

        for q in range(4):
            for rdma in r1[q]:
                rdma.wait()
            acc_ref[rows(q), :] = (acc_ref[rows(q), :]
                                   + rctx_ref[q].astype(jnp.float32))
            l_ref[rows(q), :] = l_ref[rows(q), :] + rl_ref[q]
            pack2_ref[rows(q), :] = acc_ref[rows(q), :].astype(jnp.bfloat16)
            lp_ref[1, rows(q), :] = l_ref[rows(q), :]
            r2[q] = [exchange(pack2_ref.at[rows(q)], rctx_ref.at[4 + q],
                              link2[q], 4 + q),
                     exchange(lp_ref.at[1, rows(q)], rl_ref.at[4 + q],
                              link2[q], 12 + q)]
            for rdma in r2[q]:
                rdma.start()

        ldw.wait()
        wob = wo_ref[...].astype(jnp.bfloat16)

        for q in range(4):
            for rdma in r2[q]:
                rdma.wait()
            acc_q = acc_ref[rows(q), :] + rctx_ref[4 + q].astype(jnp.float32)
            l_q = l_ref[rows(q), :] + rl_ref[4 + q]
            ctx_q = (acc_q.reshape(G, HQ, DH) / l_q[:, :, None]).reshape(G, D)
            out_q = lax.dot_general(
                ctx_q.astype(jnp.bfloat16), wob, (((1,), (0,)), ((), ())),
                preferred_element_type=jnp.float32)
            for a in range(4):
                out_ref[0, 256 * a + 64 * q:256 * a + 64 * (q + 1), :] = \
                    out_q[64 * a:64 * (a + 1), :]

    return pl.pallas_call(
        body,
        out_shape=jax.ShapeDtypeStruct((1, SQ, D), jnp.float32),
        in_specs=[
            pl.BlockSpec(memory_space=pltpu.MemorySpace.VMEM),
            pl.BlockSpec(memory_space=pltpu.MemorySpace.VMEM),
            pl.BlockSpec(memory_space=pltpu.MemorySpace.HBM),
            pl.BlockSpec(memory_space=pltpu.MemorySpace.HBM),
            pl.BlockSpec(memory_space=pltpu.MemorySpace.HBM),
        ],
        out_specs=pl.BlockSpec(memory_space=pltpu.MemorySpace.VMEM),
        scratch_shapes=[
            pltpu.VMEM((1, SKV, HQ, DH), jnp.float32),
            pltpu.VMEM((1, SKV, HQ, DH), jnp.float32),
            pltpu.VMEM((D, D), jnp.float32),
            pltpu.VMEM((SQ, D), jnp.float32),
            pltpu.VMEM((SQ, HQ), jnp.float32),
            pltpu.VMEM((SQ, D), jnp.bfloat16),
            pltpu.VMEM((SQ, D), jnp.bfloat16),
            pltpu.VMEM((2, SQ, HQ), jnp.float32),
            pltpu.VMEM((8, G, D), jnp.bfloat16),
            pltpu.VMEM((8, G, HQ), jnp.float32),
            pltpu.SemaphoreType.DMA((3,)),
            pltpu.SemaphoreType.DMA((16,)),
            pltpu.SemaphoreType.DMA((16,)),
        ],
        compiler_params=pltpu.CompilerParams(
            collective_id=0, vmem_limit_bytes=64 * 1024 * 1024),
    )(x, Wq, K_ext, V_ext, Wo)
